# baseline (speedup 1.0000x reference)
import functools

import jax
import jax.numpy as jnp
from jax import lax
from jax.experimental import pallas as pl
from jax.experimental.pallas import tpu as pltpu
from jax.experimental.pallas import tpu_sc as plsc

E = 8
TOP_K = 2
D = 1024
FF = 3584
T = 2048
TK = T * TOP_K
B = 256
NB = (TK + E * B) // B
NPAD = NB * B
FFB = 1792
NFF = FF // FFB

NW = 32
NC = 2
DW = D // 2


def _router_body(x_ref, wg_ref, i1_ref, i2_ref, wa_ref, wb_ref):
    x = x_ref[...]
    wg = wg_ref[...]
    logits = lax.dot_general(x, wg, (((1,), (1,)), ((), ())),
                             preferred_element_type=jnp.float32)
    ii = lax.broadcasted_iota(jnp.int32, (T, E), 1)
    m1 = jnp.max(logits, axis=1, keepdims=True)
    i1 = jnp.min(jnp.where(logits == m1, ii, E), axis=1, keepdims=True)
    masked = jnp.where(ii == i1, -jnp.inf, logits)
    m2 = jnp.max(masked, axis=1, keepdims=True)
    i2 = jnp.min(jnp.where(masked == m2, ii, E), axis=1, keepdims=True)
    wa = 1.0 / (1.0 + jnp.exp(m2 - m1))
    i1_ref[...] = i1
    i2_ref[...] = i2
    wa_ref[...] = wa
    wb_ref[...] = 1.0 - wa


def _router(x, w_gate):
    return pl.pallas_call(
        _router_body,
        out_shape=(
            jax.ShapeDtypeStruct((T, 1), jnp.int32),
            jax.ShapeDtypeStruct((T, 1), jnp.int32),
            jax.ShapeDtypeStruct((T, 1), jnp.float32),
            jax.ShapeDtypeStruct((T, 1), jnp.float32),
        ),
    )(x, w_gate)


def _dispatch_meta(i1, i2, wa, wb):
    e_f = jnp.stack([i1, i2], axis=1).reshape(TK)
    oh = (e_f[:, None] == jnp.arange(E, dtype=jnp.int32)[None, :]).astype(jnp.int32)
    csum = jnp.cumsum(oh, axis=0)
    counts = csum[-1]
    rank = jnp.take_along_axis(csum, e_f[:, None], axis=1)[:, 0] - 1
    padded = ((counts + B - 1) // B) * B
    gend = jnp.cumsum(padded)
    gstart = gend - padded
    dest = (gstart[e_f] + rank).astype(jnp.int32)
    tok = jnp.arange(TK, dtype=jnp.int32) // TOP_K
    src_tok = jnp.zeros((NPAD,), jnp.int32).at[dest].set(tok)
    w_f = jnp.stack([wa, wb], axis=1).reshape(TK)
    wsort = jnp.zeros((NPAD,), jnp.float32).at[dest].set(w_f)
    pos0 = dest[0::2]
    pos1 = dest[1::2]
    bidx = jnp.arange(NB, dtype=jnp.int32)
    ends_b = (gend // B).astype(jnp.int32)
    be_raw = jnp.sum((bidx[:, None] >= ends_b[None, :]).astype(jnp.int32),
                     axis=1)
    return src_tok, wsort, pos0, pos1, be_raw


GCH = 96


def _sc_gather(x3, src_tok):
    mesh = plsc.VectorSubcoreMesh(core_axis_name="c", subcore_axis_name="s")
    rows_per_w = NPAD // NW

    @functools.partial(
        pl.kernel,
        mesh=mesh,
        out_type=jax.ShapeDtypeStruct((NPAD, DW), jnp.int32),
        scratch_types=[
            pltpu.VMEM((GCH,), jnp.int32),
            pltpu.VMEM((GCH,), jnp.int32),
            pltpu.VMEM((GCH, DW), jnp.int32),
            pltpu.VMEM((GCH, DW), jnp.int32),
            pltpu.SemaphoreType.DMA,
            pltpu.SemaphoreType.DMA,
            pltpu.SemaphoreType.DMA,
            pltpu.SemaphoreType.DMA,
        ],
    )
    def k(x_hbm, idx_hbm, out_hbm, i0_v, i1_v, r0_v, r1_v, s0, s1, t0, t1):
        wid = lax.axis_index("s") * NC + lax.axis_index("c")
        b0 = wid * rows_per_w
        b1 = b0 + GCH
        pltpu.sync_copy(idx_hbm.at[pl.ds(b0, GCH)], i0_v)
        g0 = pltpu.async_copy(x_hbm.at[i0_v], r0_v, s0)
        pltpu.sync_copy(idx_hbm.at[pl.ds(b1, GCH)], i1_v)
        g1 = pltpu.async_copy(x_hbm.at[i1_v], r1_v, s1)
        g0.wait()
        st0 = pltpu.async_copy(r0_v, out_hbm.at[pl.ds(b0, GCH)], t0)
        g1.wait()
        st1 = pltpu.async_copy(r1_v, out_hbm.at[pl.ds(b1, GCH)], t1)
        st0.wait()
        st1.wait()

    return k(x3, src_tok)


def _ffn1_body(be_ref, x_ref, w1_ref, w3_ref, h_ref, w1s, w3s):
    b = pl.program_id(1)

    @pl.when(be_ref[b] < E)
    def _():
        changed = jnp.logical_or(
            b == 0, be_ref[b] != be_ref[jnp.maximum(b - 1, 0)])

        @pl.when(changed)
        def _():
            w1s[...] = w1_ref[0].astype(jnp.bfloat16)
            w3s[...] = w3_ref[0].astype(jnp.bfloat16)

        xb = x_ref[...]
        g = lax.dot_general(xb, w1s[...], (((1,), (1,)), ((), ())),
                            preferred_element_type=jnp.float32)
        u = lax.dot_general(xb, w3s[...], (((1,), (1,)), ((), ())),
                            preferred_element_type=jnp.float32)
        sg = 1.0 / (1.0 + jnp.exp(-g))
        h_ref[...] = (g * sg * u).astype(jnp.bfloat16)


def _ffn1(be_raw, x_sorted, w1, w3):
    grid_spec = pltpu.PrefetchScalarGridSpec(
        num_scalar_prefetch=1,
        grid=(NFF, NB),
        in_specs=[
            pl.BlockSpec((B, D), lambda f, b, be: (b, 0)),
            pl.BlockSpec((1, FFB, D),
                         lambda f, b, be: (jnp.minimum(be[b], E - 1), f, 0)),
            pl.BlockSpec((1, FFB, D),
                         lambda f, b, be: (jnp.minimum(be[b], E - 1), f, 0)),
        ],
        out_specs=pl.BlockSpec((B, FFB), lambda f, b, be: (b, f)),
        scratch_shapes=[
            pltpu.VMEM((FFB, D), jnp.bfloat16),
            pltpu.VMEM((FFB, D), jnp.bfloat16),
        ],
    )
    return pl.pallas_call(
        _ffn1_body,
        grid_spec=grid_spec,
        out_shape=jax.ShapeDtypeStruct((NPAD, FF), jnp.bfloat16),
        compiler_params=pltpu.CompilerParams(
            dimension_semantics=("arbitrary", "arbitrary")),
    )(be_raw, x_sorted, w1, w3)


def _ffn2_body(be_ref, h_ref, w2_ref, ws_ref, y_ref, w2s):
    b = pl.program_id(0)

    @pl.when(be_ref[b] < E)
    def _():
        changed = jnp.logical_or(
            b == 0, be_ref[b] != be_ref[jnp.maximum(b - 1, 0)])

        @pl.when(changed)
        def _():
            w2s[...] = w2_ref[0].astype(jnp.bfloat16)

        out = lax.dot_general(h_ref[...], w2s[...], (((1,), (1,)), ((), ())),
                              preferred_element_type=jnp.float32)
        y_ref[...] = out * ws_ref[:, 0:1]


def _ffn2(be_raw, h, w2, w_bcast):
    grid_spec = pltpu.PrefetchScalarGridSpec(
        num_scalar_prefetch=1,
        grid=(NB,),
        in_specs=[
            pl.BlockSpec((B, FF), lambda b, be: (b, 0)),
            pl.BlockSpec((1, D, FF),
                         lambda b, be: (jnp.minimum(be[b], E - 1), 0, 0)),
            pl.BlockSpec((B, 128), lambda b, be: (b, 0)),
        ],
        out_specs=pl.BlockSpec((B, D), lambda b, be: (b, 0)),
        scratch_shapes=[
            pltpu.VMEM((D, FF), jnp.bfloat16),
        ],
    )
    return pl.pallas_call(
        _ffn2_body,
        grid_spec=grid_spec,
        out_shape=jax.ShapeDtypeStruct((NPAD, D), jnp.float32),
        compiler_params=pltpu.CompilerParams(
            dimension_semantics=("arbitrary",)),
    )(be_raw, h, w2, w_bcast)


CCH = 32


def _sc_combine(y, pos0, pos1):
    mesh = plsc.VectorSubcoreMesh(core_axis_name="c", subcore_axis_name="s")
    tok_per_w = T // NW

    @functools.partial(
        pl.kernel,
        mesh=mesh,
        out_type=jax.ShapeDtypeStruct((T, D), jnp.float32),
        scratch_types=[
            pltpu.VMEM((CCH,), jnp.int32),
            pltpu.VMEM((CCH,), jnp.int32),
            pltpu.VMEM((CCH, D), jnp.float32),
            pltpu.VMEM((CCH, D), jnp.float32),
            pltpu.SemaphoreType.DMA,
            pltpu.SemaphoreType.DMA,
        ],
    )
    def k(y_hbm, p0_hbm, p1_hbm, out_hbm, i0_v, i1_v, r0_v, r1_v, s0, s1):
        wid = lax.axis_index("s") * NC + lax.axis_index("c")
        base = wid * tok_per_w
        for t in range(tok_per_w // CCH):
            b0 = base + t * CCH
            pltpu.sync_copy(p0_hbm.at[pl.ds(b0, CCH)], i0_v)
            pltpu.sync_copy(p1_hbm.at[pl.ds(b0, CCH)], i1_v)
            cp0 = pltpu.async_copy(y_hbm.at[i0_v], r0_v, s0)
            cp1 = pltpu.async_copy(y_hbm.at[i1_v], r1_v, s1)
            cp0.wait()
            cp1.wait()

            def body(i, _):
                r = i // (D // 16)
                c = (i % (D // 16)) * 16
                r0_v[r, pl.ds(c, 16)] = (r0_v[r, pl.ds(c, 16)]
                                         + r1_v[r, pl.ds(c, 16)])
                return 0

            lax.fori_loop(0, CCH * (D // 16), body, 0)
            pltpu.sync_copy(r0_v, out_hbm.at[pl.ds(b0, CCH)])

    return k(y, pos0, pos1)


def kernel(hidden_states, w_gate, w1, w3, w2):
    orig_shape = hidden_states.shape
    x = hidden_states.reshape(T, D)
    i1, i2, wa, wb = _router(x, w_gate)
    src_tok, wsort, pos0, pos1, be_raw = _dispatch_meta(
        i1[:, 0], i2[:, 0], wa[:, 0], wb[:, 0])
    xw = lax.bitcast_convert_type(
        x.astype(jnp.bfloat16).reshape(T, DW, 2), jnp.int32)
    x_sorted = lax.bitcast_convert_type(
        _sc_gather(xw, src_tok), jnp.bfloat16).reshape(NPAD, D)
    h = _ffn1(be_raw, x_sorted, w1, w3)
    w_bcast = jnp.broadcast_to(wsort[:, None], (NPAD, 128))
    y = _ffn2(be_raw, h, w2, w_bcast)
    final = _sc_combine(y, pos0, pos1)
    return final.reshape(orig_shape)


# speedup vs baseline: 1.1339x; 1.1339x over previous
import functools

import jax
import jax.numpy as jnp
from jax import lax
from jax.experimental import pallas as pl
from jax.experimental.pallas import tpu as pltpu
from jax.experimental.pallas import tpu_sc as plsc

E = 8
TOP_K = 2
D = 1024
FF = 3584
T = 2048
TK = T * TOP_K
B = 256
NB = (TK + E * B) // B
NPAD = NB * B
FFB = 1792
NFF = FF // FFB

NW = 32
NC = 2
DW = D // 2


def _router_body(x_ref, wg_ref, dest_ref, wp_ref, be_ref):
    x = x_ref[...]
    wg = wg_ref[...]
    logits = lax.dot_general(x, wg, (((1,), (1,)), ((), ())),
                             preferred_element_type=jnp.float32)
    ii = lax.broadcasted_iota(jnp.int32, (T, E), 1)
    m1 = jnp.max(logits, axis=1, keepdims=True)
    i1 = jnp.min(jnp.where(logits == m1, ii, E), axis=1, keepdims=True)
    masked = jnp.where(ii == i1, -jnp.inf, logits)
    m2 = jnp.max(masked, axis=1, keepdims=True)
    i2 = jnp.min(jnp.where(masked == m2, ii, E), axis=1, keepdims=True)
    wa = 1.0 / (1.0 + jnp.exp(m2 - m1))

    oh0 = (i1 == ii).astype(jnp.float32)
    oh1 = (i2 == ii).astype(jnp.float32)
    rowsum = oh0 + oh1
    rr = lax.broadcasted_iota(jnp.int32, (T, T), 0)
    cc = lax.broadcasted_iota(jnp.int32, (T, T), 1)
    lstrict = (cc < rr).astype(jnp.float32)
    a_excl = lax.dot_general(lstrict, rowsum, (((1,), (0,)), ((), ())),
                             preferred_element_type=jnp.float32)
    counts = jnp.sum(rowsum, axis=0, keepdims=True)
    padded = jnp.bitwise_and(counts.astype(jnp.int32) + (B - 1),
                             -B).astype(jnp.float32)
    e1 = lax.broadcasted_iota(jnp.int32, (E, E), 0)
    e2 = lax.broadcasted_iota(jnp.int32, (E, E), 1)
    u8 = (e1 <= e2).astype(jnp.float32)
    gend = lax.dot_general(padded, u8, (((1,), (0,)), ((), ())),
                           preferred_element_type=jnp.float32)
    gstart = gend - padded
    base0 = jnp.sum(oh0 * gstart, axis=1, keepdims=True)
    base1 = jnp.sum(oh1 * gstart, axis=1, keepdims=True)
    rank0 = jnp.sum(oh0 * a_excl, axis=1, keepdims=True)
    rank1 = jnp.sum(oh1 * a_excl, axis=1, keepdims=True)
    dest0 = (base0 + rank0).astype(jnp.int32)
    dest1 = (base1 + rank1).astype(jnp.int32)
    dest_ref[...] = jnp.concatenate([dest0, dest1], axis=1)
    wp_ref[...] = jnp.concatenate([wa, 1.0 - wa], axis=1)
    bidx = lax.broadcasted_iota(jnp.int32, (NB, E), 0).astype(jnp.float32)
    ends_b = gend * (1.0 / B)
    be = jnp.sum((bidx >= ends_b).astype(jnp.int32), axis=1, keepdims=True)
    be_ref[...] = be


def _router(x, w_gate):
    return pl.pallas_call(
        _router_body,
        out_shape=(
            jax.ShapeDtypeStruct((T, 2), jnp.int32),
            jax.ShapeDtypeStruct((T, 2), jnp.float32),
            jax.ShapeDtypeStruct((NB, 1), jnp.int32),
        ),
    )(x, w_gate)


PPW = TK // NW


def _sc_dispatch(xw, dest, wf):
    mesh = plsc.VectorSubcoreMesh(core_axis_name="c", subcore_axis_name="s")

    @functools.partial(
        pl.kernel,
        mesh=mesh,
        out_type=(
            jax.ShapeDtypeStruct((NPAD, DW), jnp.int32),
            jax.ShapeDtypeStruct((NPAD,), jnp.float32),
        ),
        scratch_types=[
            pltpu.VMEM((PPW,), jnp.int32),
            pltpu.VMEM((PPW,), jnp.int32),
            pltpu.VMEM((PPW,), jnp.float32),
            pltpu.VMEM((PPW, DW), jnp.int32),
            pltpu.SemaphoreType.DMA,
            pltpu.SemaphoreType.DMA,
            pltpu.SemaphoreType.DMA,
        ],
    )
    def k(xw_hbm, dest_hbm, wf_hbm, xs_hbm, ws_hbm,
          tok_v, dst_v, wf_v, rows_v, s0, s1, s2):
        wid = lax.axis_index("s") * NC + lax.axis_index("c")
        p0 = wid * PPW
        for j in range(PPW // 16):
            v = lax.iota(jnp.int32, 16) + (16 * j)
            tok_v[pl.ds(16 * j, 16)] = lax.shift_right_logical(p0 + v, 1)
        pltpu.sync_copy(dest_hbm.at[pl.ds(p0, PPW)], dst_v)
        pltpu.sync_copy(wf_hbm.at[pl.ds(p0, PPW)], wf_v)
        g = pltpu.async_copy(xw_hbm.at[tok_v], rows_v, s0)
        g.wait()
        st = pltpu.async_copy(rows_v, xs_hbm.at[dst_v], s1)
        sw = pltpu.async_copy(wf_v, ws_hbm.at[dst_v], s2)
        st.wait()
        sw.wait()

    return k(xw, dest, wf)


def _ffn1_body(be_ref, x_ref, w1_ref, w3_ref, h_ref, w1s, w3s):
    b = pl.program_id(1)
    changed = jnp.logical_or(
        b == 0, be_ref[b] != be_ref[jnp.maximum(b - 1, 0)])

    @pl.when(changed)
    def _():
        w1s[...] = w1_ref[0].astype(jnp.bfloat16)
        w3s[...] = w3_ref[0].astype(jnp.bfloat16)

    xb = x_ref[...]
    g = lax.dot_general(xb, w1s[...], (((1,), (1,)), ((), ())),
                        preferred_element_type=jnp.float32)
    u = lax.dot_general(xb, w3s[...], (((1,), (1,)), ((), ())),
                        preferred_element_type=jnp.float32)
    sg = 1.0 / (1.0 + jnp.exp(-g))
    h_ref[...] = (g * sg * u).astype(jnp.bfloat16)


def _ffn1(be_raw, x_sorted, w1, w3):
    grid_spec = pltpu.PrefetchScalarGridSpec(
        num_scalar_prefetch=1,
        grid=(NFF, NB),
        in_specs=[
            pl.BlockSpec((B, D), lambda f, b, be: (b, 0)),
            pl.BlockSpec((1, FFB, D),
                         lambda f, b, be: (jnp.minimum(be[b], E - 1), f, 0)),
            pl.BlockSpec((1, FFB, D),
                         lambda f, b, be: (jnp.minimum(be[b], E - 1), f, 0)),
        ],
        out_specs=pl.BlockSpec((B, FFB), lambda f, b, be: (b, f)),
        scratch_shapes=[
            pltpu.VMEM((FFB, D), jnp.bfloat16),
            pltpu.VMEM((FFB, D), jnp.bfloat16),
        ],
    )
    return pl.pallas_call(
        _ffn1_body,
        grid_spec=grid_spec,
        out_shape=jax.ShapeDtypeStruct((NPAD, FF), jnp.bfloat16),
        compiler_params=pltpu.CompilerParams(
            dimension_semantics=("arbitrary", "arbitrary")),
    )(be_raw, x_sorted, w1, w3)


def _ffn2_body(be_ref, h_ref, w2_ref, ws_ref, y_ref, w2s):
    b = pl.program_id(0)
    changed = jnp.logical_or(
        b == 0, be_ref[b] != be_ref[jnp.maximum(b - 1, 0)])

    @pl.when(changed)
    def _():
        w2s[...] = w2_ref[0].astype(jnp.bfloat16)

    out = lax.dot_general(h_ref[...], w2s[...], (((1,), (1,)), ((), ())),
                          preferred_element_type=jnp.float32)
    y_ref[...] = out * ws_ref[:, 0:1]


def _ffn2(be_raw, h, w2, w_bcast):
    grid_spec = pltpu.PrefetchScalarGridSpec(
        num_scalar_prefetch=1,
        grid=(NB,),
        in_specs=[
            pl.BlockSpec((B, FF), lambda b, be: (b, 0)),
            pl.BlockSpec((1, D, FF),
                         lambda b, be: (jnp.minimum(be[b], E - 1), 0, 0)),
            pl.BlockSpec((B, 128), lambda b, be: (b, 0)),
        ],
        out_specs=pl.BlockSpec((B, D), lambda b, be: (b, 0)),
        scratch_shapes=[
            pltpu.VMEM((D, FF), jnp.bfloat16),
        ],
    )
    return pl.pallas_call(
        _ffn2_body,
        grid_spec=grid_spec,
        out_shape=jax.ShapeDtypeStruct((NPAD, D), jnp.float32),
        compiler_params=pltpu.CompilerParams(
            dimension_semantics=("arbitrary",)),
    )(be_raw, h, w2, w_bcast)


CCH = 32


def _sc_combine(y, pos0, pos1):
    mesh = plsc.VectorSubcoreMesh(core_axis_name="c", subcore_axis_name="s")
    tok_per_w = T // NW

    @functools.partial(
        pl.kernel,
        mesh=mesh,
        out_type=jax.ShapeDtypeStruct((T, D), jnp.float32),
        scratch_types=[
            pltpu.VMEM((CCH,), jnp.int32),
            pltpu.VMEM((CCH,), jnp.int32),
            pltpu.VMEM((CCH, D), jnp.float32),
            pltpu.VMEM((CCH, D), jnp.float32),
            pltpu.SemaphoreType.DMA,
            pltpu.SemaphoreType.DMA,
        ],
    )
    def k(y_hbm, p0_hbm, p1_hbm, out_hbm, i0_v, i1_v, r0_v, r1_v, s0, s1):
        wid = lax.axis_index("s") * NC + lax.axis_index("c")
        base = wid * tok_per_w
        for t in range(tok_per_w // CCH):
            b0 = base + t * CCH
            pltpu.sync_copy(p0_hbm.at[pl.ds(b0, CCH)], i0_v)
            pltpu.sync_copy(p1_hbm.at[pl.ds(b0, CCH)], i1_v)
            cp0 = pltpu.async_copy(y_hbm.at[i0_v], r0_v, s0)
            cp1 = pltpu.async_copy(y_hbm.at[i1_v], r1_v, s1)
            cp0.wait()
            cp1.wait()

            def body(i, _):
                r = i // (D // 16)
                c = (i % (D // 16)) * 16
                r0_v[r, pl.ds(c, 16)] = (r0_v[r, pl.ds(c, 16)]
                                         + r1_v[r, pl.ds(c, 16)])
                return 0

            lax.fori_loop(0, CCH * (D // 16), body, 0)
            pltpu.sync_copy(r0_v, out_hbm.at[pl.ds(b0, CCH)])

    return k(y, pos0, pos1)


def kernel(hidden_states, w_gate, w1, w3, w2):
    orig_shape = hidden_states.shape
    x = hidden_states.reshape(T, D)
    dest, wp, be_raw = _router(x, w_gate)
    xw = lax.bitcast_convert_type(
        x.astype(jnp.bfloat16).reshape(T, DW, 2), jnp.int32)
    xs_w, wsort = _sc_dispatch(xw, dest.reshape(TK), wp.reshape(TK))
    x_sorted = lax.bitcast_convert_type(
        xs_w, jnp.bfloat16).reshape(NPAD, D)
    h = _ffn1(be_raw[:, 0], x_sorted, w1, w3)
    w_bcast = jnp.broadcast_to(wsort[:, None], (NPAD, 128))
    y = _ffn2(be_raw[:, 0], h, w2, w_bcast)
    final = _sc_combine(y, dest[:, 0], dest[:, 1])
    return final.reshape(orig_shape)


# speedup vs baseline: 1.2206x; 1.0764x over previous
import functools

import jax
import jax.numpy as jnp
from jax import lax
from jax.experimental import pallas as pl
from jax.experimental.pallas import tpu as pltpu
from jax.experimental.pallas import tpu_sc as plsc

E = 8
TOP_K = 2
D = 1024
FF = 3584
T = 2048
TK = T * TOP_K
B = 512
NB = (TK + E * B) // B
NPAD = NB * B
FFB = 1792
NFF = FF // FFB

NW = 32
NC = 2
DW = D // 2


def _router_body(x_ref, wg_ref, dest_ref, wp_ref, be_ref):
    x = x_ref[...]
    wg = wg_ref[...]
    logits = lax.dot_general(x, wg, (((1,), (1,)), ((), ())),
                             preferred_element_type=jnp.float32)
    ii = lax.broadcasted_iota(jnp.int32, (T, E), 1)
    m1 = jnp.max(logits, axis=1, keepdims=True)
    i1 = jnp.min(jnp.where(logits == m1, ii, E), axis=1, keepdims=True)
    masked = jnp.where(ii == i1, -jnp.inf, logits)
    m2 = jnp.max(masked, axis=1, keepdims=True)
    i2 = jnp.min(jnp.where(masked == m2, ii, E), axis=1, keepdims=True)
    wa = 1.0 / (1.0 + jnp.exp(m2 - m1))

    oh0 = (i1 == ii).astype(jnp.float32)
    oh1 = (i2 == ii).astype(jnp.float32)
    rowsum = oh0 + oh1
    rr = lax.broadcasted_iota(jnp.int32, (T, T), 0)
    cc = lax.broadcasted_iota(jnp.int32, (T, T), 1)
    lstrict = (cc < rr).astype(jnp.float32)
    a_excl = lax.dot_general(lstrict, rowsum, (((1,), (0,)), ((), ())),
                             preferred_element_type=jnp.float32)
    counts = jnp.sum(rowsum, axis=0, keepdims=True)
    padded = jnp.bitwise_and(counts.astype(jnp.int32) + (B - 1),
                             -B).astype(jnp.float32)
    e1 = lax.broadcasted_iota(jnp.int32, (E, E), 0)
    e2 = lax.broadcasted_iota(jnp.int32, (E, E), 1)
    u8 = (e1 <= e2).astype(jnp.float32)
    gend = lax.dot_general(padded, u8, (((1,), (0,)), ((), ())),
                           preferred_element_type=jnp.float32)
    gstart = gend - padded
    base0 = jnp.sum(oh0 * gstart, axis=1, keepdims=True)
    base1 = jnp.sum(oh1 * gstart, axis=1, keepdims=True)
    rank0 = jnp.sum(oh0 * a_excl, axis=1, keepdims=True)
    rank1 = jnp.sum(oh1 * a_excl, axis=1, keepdims=True)
    dest0 = (base0 + rank0).astype(jnp.int32)
    dest1 = (base1 + rank1).astype(jnp.int32)
    dest_ref[...] = jnp.concatenate([dest0, dest1], axis=1)
    wp_ref[...] = jnp.concatenate([wa, 1.0 - wa], axis=1)
    bidx = lax.broadcasted_iota(jnp.int32, (NB, E), 0).astype(jnp.float32)
    ends_b = gend * (1.0 / B)
    be = jnp.sum((bidx >= ends_b).astype(jnp.int32), axis=1, keepdims=True)
    be_ref[...] = be


def _router(x, w_gate):
    return pl.pallas_call(
        _router_body,
        out_shape=(
            jax.ShapeDtypeStruct((T, 2), jnp.int32),
            jax.ShapeDtypeStruct((T, 2), jnp.float32),
            jax.ShapeDtypeStruct((NB, 1), jnp.int32),
        ),
    )(x, w_gate)


PPW = TK // NW


def _sc_dispatch(xw, dest, wf):
    mesh = plsc.VectorSubcoreMesh(core_axis_name="c", subcore_axis_name="s")

    @functools.partial(
        pl.kernel,
        mesh=mesh,
        out_type=(
            jax.ShapeDtypeStruct((NPAD, DW), jnp.int32),
            jax.ShapeDtypeStruct((NPAD,), jnp.float32),
        ),
        scratch_types=[
            pltpu.VMEM((PPW,), jnp.int32),
            pltpu.VMEM((PPW,), jnp.int32),
            pltpu.VMEM((PPW,), jnp.float32),
            pltpu.VMEM((PPW, DW), jnp.int32),
            pltpu.SemaphoreType.DMA,
            pltpu.SemaphoreType.DMA,
            pltpu.SemaphoreType.DMA,
        ],
    )
    def k(xw_hbm, dest_hbm, wf_hbm, xs_hbm, ws_hbm,
          tok_v, dst_v, wf_v, rows_v, s0, s1, s2):
        wid = lax.axis_index("s") * NC + lax.axis_index("c")
        p0 = wid * PPW
        for j in range(PPW // 16):
            v = lax.iota(jnp.int32, 16) + (16 * j)
            tok_v[pl.ds(16 * j, 16)] = lax.shift_right_logical(p0 + v, 1)
        pltpu.sync_copy(dest_hbm.at[pl.ds(p0, PPW)], dst_v)
        pltpu.sync_copy(wf_hbm.at[pl.ds(p0, PPW)], wf_v)
        g = pltpu.async_copy(xw_hbm.at[tok_v], rows_v, s0)
        g.wait()
        st = pltpu.async_copy(rows_v, xs_hbm.at[dst_v], s1)
        sw = pltpu.async_copy(wf_v, ws_hbm.at[dst_v], s2)
        st.wait()
        sw.wait()

    return k(xw, dest, wf)


def _ffn1_body(be_ref, x_ref, w1_ref, w3_ref, h_ref, w1s, w3s):
    b = pl.program_id(1)
    changed = jnp.logical_or(
        b == 0, be_ref[b] != be_ref[jnp.maximum(b - 1, 0)])

    @pl.when(changed)
    def _():
        w1s[...] = w1_ref[0].astype(jnp.bfloat16)
        w3s[...] = w3_ref[0].astype(jnp.bfloat16)

    xb = x_ref[...]
    g = lax.dot_general(xb, w1s[...], (((1,), (1,)), ((), ())),
                        preferred_element_type=jnp.float32)
    u = lax.dot_general(xb, w3s[...], (((1,), (1,)), ((), ())),
                        preferred_element_type=jnp.float32)
    sg = 1.0 / (1.0 + jnp.exp(-g))
    h_ref[...] = (g * sg * u).astype(jnp.bfloat16)


def _ffn1(be_raw, x_sorted, w1, w3):
    grid_spec = pltpu.PrefetchScalarGridSpec(
        num_scalar_prefetch=1,
        grid=(NFF, NB),
        in_specs=[
            pl.BlockSpec((B, D), lambda f, b, be: (b, 0)),
            pl.BlockSpec((1, FFB, D),
                         lambda f, b, be: (jnp.minimum(be[b], E - 1), f, 0)),
            pl.BlockSpec((1, FFB, D),
                         lambda f, b, be: (jnp.minimum(be[b], E - 1), f, 0)),
        ],
        out_specs=pl.BlockSpec((B, FFB), lambda f, b, be: (b, f)),
        scratch_shapes=[
            pltpu.VMEM((FFB, D), jnp.bfloat16),
            pltpu.VMEM((FFB, D), jnp.bfloat16),
        ],
    )
    return pl.pallas_call(
        _ffn1_body,
        grid_spec=grid_spec,
        out_shape=jax.ShapeDtypeStruct((NPAD, FF), jnp.bfloat16),
        compiler_params=pltpu.CompilerParams(
            dimension_semantics=("arbitrary", "arbitrary")),
    )(be_raw, x_sorted, w1, w3)


def _ffn2_body(be_ref, h_ref, w2_ref, ws_ref, y_ref, w2s):
    b = pl.program_id(0)
    changed = jnp.logical_or(
        b == 0, be_ref[b] != be_ref[jnp.maximum(b - 1, 0)])

    @pl.when(changed)
    def _():
        w2s[...] = w2_ref[0].astype(jnp.bfloat16)

    out = lax.dot_general(h_ref[...], w2s[...], (((1,), (1,)), ((), ())),
                          preferred_element_type=jnp.float32)
    y_ref[...] = out * ws_ref[:, 0:1]


def _ffn2(be_raw, h, w2, w_bcast):
    grid_spec = pltpu.PrefetchScalarGridSpec(
        num_scalar_prefetch=1,
        grid=(NB,),
        in_specs=[
            pl.BlockSpec((B, FF), lambda b, be: (b, 0)),
            pl.BlockSpec((1, D, FF),
                         lambda b, be: (jnp.minimum(be[b], E - 1), 0, 0)),
            pl.BlockSpec((B, 128), lambda b, be: (b, 0)),
        ],
        out_specs=pl.BlockSpec((B, D), lambda b, be: (b, 0)),
        scratch_shapes=[
            pltpu.VMEM((D, FF), jnp.bfloat16),
        ],
    )
    return pl.pallas_call(
        _ffn2_body,
        grid_spec=grid_spec,
        out_shape=jax.ShapeDtypeStruct((NPAD, D), jnp.float32),
        compiler_params=pltpu.CompilerParams(
            dimension_semantics=("arbitrary",)),
    )(be_raw, h, w2, w_bcast)


CCH = 32


def _sc_combine(y, pos0, pos1):
    mesh = plsc.VectorSubcoreMesh(core_axis_name="c", subcore_axis_name="s")
    tok_per_w = T // NW

    @functools.partial(
        pl.kernel,
        mesh=mesh,
        out_type=jax.ShapeDtypeStruct((T, D), jnp.float32),
        scratch_types=[
            pltpu.VMEM((CCH,), jnp.int32),
            pltpu.VMEM((CCH,), jnp.int32),
            pltpu.VMEM((CCH, D), jnp.float32),
            pltpu.VMEM((CCH, D), jnp.float32),
            pltpu.SemaphoreType.DMA,
            pltpu.SemaphoreType.DMA,
        ],
    )
    def k(y_hbm, p0_hbm, p1_hbm, out_hbm, i0_v, i1_v, r0_v, r1_v, s0, s1):
        wid = lax.axis_index("s") * NC + lax.axis_index("c")
        base = wid * tok_per_w
        for t in range(tok_per_w // CCH):
            b0 = base + t * CCH
            pltpu.sync_copy(p0_hbm.at[pl.ds(b0, CCH)], i0_v)
            pltpu.sync_copy(p1_hbm.at[pl.ds(b0, CCH)], i1_v)
            cp0 = pltpu.async_copy(y_hbm.at[i0_v], r0_v, s0)
            cp1 = pltpu.async_copy(y_hbm.at[i1_v], r1_v, s1)
            cp0.wait()
            cp1.wait()

            def body(r, _):
                for c in range(D // 16):
                    r0_v[r, pl.ds(c * 16, 16)] = (
                        r0_v[r, pl.ds(c * 16, 16)]
                        + r1_v[r, pl.ds(c * 16, 16)])
                return 0

            lax.fori_loop(0, CCH, body, 0)
            pltpu.sync_copy(r0_v, out_hbm.at[pl.ds(b0, CCH)])

    return k(y, pos0, pos1)


def kernel(hidden_states, w_gate, w1, w3, w2):
    orig_shape = hidden_states.shape
    x = hidden_states.reshape(T, D)
    dest, wp, be_raw = _router(x, w_gate)
    xw = lax.bitcast_convert_type(
        x.astype(jnp.bfloat16).reshape(T, DW, 2), jnp.int32)
    xs_w, wsort = _sc_dispatch(xw, dest.reshape(TK), wp.reshape(TK))
    x_sorted = lax.bitcast_convert_type(
        xs_w, jnp.bfloat16).reshape(NPAD, D)
    h = _ffn1(be_raw[:, 0], x_sorted, w1, w3)
    w_bcast = jnp.broadcast_to(wsort[:, None], (NPAD, 128))
    y = _ffn2(be_raw[:, 0], h, w2, w_bcast)
    final = _sc_combine(y, dest[:, 0], dest[:, 1])
    return final.reshape(orig_shape)


# speedup vs baseline: 1.3177x; 1.0796x over previous
import functools

import jax
import jax.numpy as jnp
from jax import lax
from jax.experimental import pallas as pl
from jax.experimental.pallas import tpu as pltpu
from jax.experimental.pallas import tpu_sc as plsc

E = 8
TOP_K = 2
D = 1024
FF = 3584
T = 2048
TK = T * TOP_K
B = 512
NB = (TK + E * B) // B
NPAD = NB * B
FFB = 1792
NFF = FF // FFB

NW = 32
NC = 2
DW = D // 2


def _router_body(x_ref, wg_ref, dest_ref, wp_ref, be_ref):
    x = x_ref[...]
    wg = wg_ref[...]
    logits = lax.dot_general(x, wg, (((1,), (1,)), ((), ())),
                             preferred_element_type=jnp.float32)
    ii = lax.broadcasted_iota(jnp.int32, (T, E), 1)
    m1 = jnp.max(logits, axis=1, keepdims=True)
    i1 = jnp.min(jnp.where(logits == m1, ii, E), axis=1, keepdims=True)
    masked = jnp.where(ii == i1, -jnp.inf, logits)
    m2 = jnp.max(masked, axis=1, keepdims=True)
    i2 = jnp.min(jnp.where(masked == m2, ii, E), axis=1, keepdims=True)
    wa = 1.0 / (1.0 + jnp.exp(m2 - m1))

    oh0 = (i1 == ii).astype(jnp.float32)
    oh1 = (i2 == ii).astype(jnp.float32)
    rowsum = oh0 + oh1
    rr = lax.broadcasted_iota(jnp.int32, (T, T), 0)
    cc = lax.broadcasted_iota(jnp.int32, (T, T), 1)
    lstrict = (cc < rr).astype(jnp.float32)
    a_excl = lax.dot_general(lstrict, rowsum, (((1,), (0,)), ((), ())),
                             preferred_element_type=jnp.float32)
    counts = jnp.sum(rowsum, axis=0, keepdims=True)
    padded = jnp.bitwise_and(counts.astype(jnp.int32) + (B - 1),
                             -B).astype(jnp.float32)
    e1 = lax.broadcasted_iota(jnp.int32, (E, E), 0)
    e2 = lax.broadcasted_iota(jnp.int32, (E, E), 1)
    u8 = (e1 <= e2).astype(jnp.float32)
    gend = lax.dot_general(padded, u8, (((1,), (0,)), ((), ())),
                           preferred_element_type=jnp.float32)
    gstart = gend - padded
    base0 = jnp.sum(oh0 * gstart, axis=1, keepdims=True)
    base1 = jnp.sum(oh1 * gstart, axis=1, keepdims=True)
    rank0 = jnp.sum(oh0 * a_excl, axis=1, keepdims=True)
    rank1 = jnp.sum(oh1 * a_excl, axis=1, keepdims=True)
    dest0 = (base0 + rank0).astype(jnp.int32)
    dest1 = (base1 + rank1).astype(jnp.int32)
    dest_ref[...] = jnp.concatenate([dest0, dest1], axis=1)
    wp_ref[...] = jnp.concatenate([wa, 1.0 - wa], axis=1)
    bidx = lax.broadcasted_iota(jnp.int32, (NB, E), 0).astype(jnp.float32)
    ends_b = gend * (1.0 / B)
    be = jnp.sum((bidx >= ends_b).astype(jnp.int32), axis=1, keepdims=True)
    be_ref[...] = be


def _router(x, w_gate):
    return pl.pallas_call(
        _router_body,
        out_shape=(
            jax.ShapeDtypeStruct((T, 2), jnp.int32),
            jax.ShapeDtypeStruct((T, 2), jnp.float32),
            jax.ShapeDtypeStruct((NB, 1), jnp.int32),
        ),
    )(x, w_gate)


PPW = TK // NW


def _sc_dispatch(xw, dest, wf):
    mesh = plsc.VectorSubcoreMesh(core_axis_name="c", subcore_axis_name="s")

    @functools.partial(
        pl.kernel,
        mesh=mesh,
        out_type=(
            jax.ShapeDtypeStruct((NPAD, DW), jnp.int32),
            jax.ShapeDtypeStruct((NPAD,), jnp.float32),
        ),
        scratch_types=[
            pltpu.VMEM((PPW,), jnp.int32),
            pltpu.VMEM((PPW,), jnp.int32),
            pltpu.VMEM((PPW,), jnp.float32),
            pltpu.VMEM((PPW, DW), jnp.int32),
            pltpu.SemaphoreType.DMA,
            pltpu.SemaphoreType.DMA,
            pltpu.SemaphoreType.DMA,
        ],
    )
    def k(xw_hbm, dest_hbm, wf_hbm, xs_hbm, ws_hbm,
          tok_v, dst_v, wf_v, rows_v, s0, s1, s2):
        wid = lax.axis_index("s") * NC + lax.axis_index("c")
        p0 = wid * PPW
        for j in range(PPW // 16):
            v = lax.iota(jnp.int32, 16) + (16 * j)
            tok_v[pl.ds(16 * j, 16)] = lax.shift_right_logical(p0 + v, 1)
        pltpu.sync_copy(dest_hbm.at[pl.ds(p0, PPW)], dst_v)
        pltpu.sync_copy(wf_hbm.at[pl.ds(p0, PPW)], wf_v)
        g = pltpu.async_copy(xw_hbm.at[tok_v], rows_v, s0)
        g.wait()
        st = pltpu.async_copy(rows_v, xs_hbm.at[dst_v], s1)
        sw = pltpu.async_copy(wf_v, ws_hbm.at[dst_v], s2)
        st.wait()
        sw.wait()

    return k(xw, dest, wf)


def _ffn1_body(be_ref, x_ref, w1_ref, w3_ref, h_ref, w1s, w3s):
    b = pl.program_id(1)

    @pl.when(be_ref[b] < E)
    def _():
        changed = jnp.logical_or(
            b == 0, be_ref[b] != be_ref[jnp.maximum(b - 1, 0)])

        @pl.when(changed)
        def _():
            w1s[...] = w1_ref[0].astype(jnp.bfloat16)
            w3s[...] = w3_ref[0].astype(jnp.bfloat16)

        xb = x_ref[...]
        g = lax.dot_general(xb, w1s[...], (((1,), (1,)), ((), ())),
                            preferred_element_type=jnp.float32)
        u = lax.dot_general(xb, w3s[...], (((1,), (1,)), ((), ())),
                            preferred_element_type=jnp.float32)
        sg = 1.0 / (1.0 + jnp.exp(-g))
        h_ref[...] = (g * sg * u).astype(jnp.bfloat16)


def _ffn1(be_raw, x_sorted, w1, w3):
    grid_spec = pltpu.PrefetchScalarGridSpec(
        num_scalar_prefetch=1,
        grid=(NFF, NB),
        in_specs=[
            pl.BlockSpec((B, D), lambda f, b, be: (b, 0)),
            pl.BlockSpec((1, FFB, D),
                         lambda f, b, be: (jnp.minimum(be[b], E - 1), f, 0)),
            pl.BlockSpec((1, FFB, D),
                         lambda f, b, be: (jnp.minimum(be[b], E - 1), f, 0)),
        ],
        out_specs=pl.BlockSpec((B, FFB), lambda f, b, be: (b, f)),
        scratch_shapes=[
            pltpu.VMEM((FFB, D), jnp.bfloat16),
            pltpu.VMEM((FFB, D), jnp.bfloat16),
        ],
    )
    return pl.pallas_call(
        _ffn1_body,
        grid_spec=grid_spec,
        out_shape=jax.ShapeDtypeStruct((NPAD, FF), jnp.bfloat16),
        compiler_params=pltpu.CompilerParams(
            dimension_semantics=("arbitrary", "arbitrary")),
    )(be_raw, x_sorted, w1, w3)


def _ffn2_body(be_ref, h_ref, w2_ref, ws_ref, y_ref, w2s):
    b = pl.program_id(0)

    @pl.when(be_ref[b] < E)
    def _():
        changed = jnp.logical_or(
            b == 0, be_ref[b] != be_ref[jnp.maximum(b - 1, 0)])

        @pl.when(changed)
        def _():
            w2s[...] = w2_ref[0].astype(jnp.bfloat16)

        out = lax.dot_general(h_ref[...], w2s[...],
                              (((1,), (1,)), ((), ())),
                              preferred_element_type=jnp.float32)
        y_ref[...] = out * ws_ref[:, 0:1]


def _ffn2(be_raw, h, w2, w_bcast):
    grid_spec = pltpu.PrefetchScalarGridSpec(
        num_scalar_prefetch=1,
        grid=(NB,),
        in_specs=[
            pl.BlockSpec((B, FF), lambda b, be: (b, 0)),
            pl.BlockSpec((1, D, FF),
                         lambda b, be: (jnp.minimum(be[b], E - 1), 0, 0)),
            pl.BlockSpec((B, 128), lambda b, be: (b, 0)),
        ],
        out_specs=pl.BlockSpec((B, D), lambda b, be: (b, 0)),
        scratch_shapes=[
            pltpu.VMEM((D, FF), jnp.bfloat16),
        ],
    )
    return pl.pallas_call(
        _ffn2_body,
        grid_spec=grid_spec,
        out_shape=jax.ShapeDtypeStruct((NPAD, D), jnp.float32),
        compiler_params=pltpu.CompilerParams(
            dimension_semantics=("arbitrary",)),
    )(be_raw, h, w2, w_bcast)


CCH = 32


def _sc_combine(y, pos0, pos1):
    mesh = plsc.VectorSubcoreMesh(core_axis_name="c", subcore_axis_name="s")
    tok_per_w = T // NW

    @functools.partial(
        pl.kernel,
        mesh=mesh,
        out_type=jax.ShapeDtypeStruct((T, D), jnp.float32),
        scratch_types=[
            pltpu.VMEM((CCH,), jnp.int32),
            pltpu.VMEM((CCH,), jnp.int32),
            pltpu.VMEM((CCH, D), jnp.float32),
            pltpu.VMEM((CCH, D), jnp.float32),
            pltpu.SemaphoreType.DMA,
            pltpu.SemaphoreType.DMA,
        ],
    )
    def k(y_hbm, p0_hbm, p1_hbm, out_hbm, i0_v, i1_v, r0_v, r1_v, s0, s1):
        wid = lax.axis_index("s") * NC + lax.axis_index("c")
        base = wid * tok_per_w
        for t in range(tok_per_w // CCH):
            b0 = base + t * CCH
            pltpu.sync_copy(p0_hbm.at[pl.ds(b0, CCH)], i0_v)
            pltpu.sync_copy(p1_hbm.at[pl.ds(b0, CCH)], i1_v)
            cp0 = pltpu.async_copy(y_hbm.at[i0_v], r0_v, s0)
            cp1 = pltpu.async_copy(y_hbm.at[i1_v], r1_v, s1)
            cp0.wait()
            cp1.wait()

            def body(r, _):
                for c in range(D // 16):
                    r0_v[r, pl.ds(c * 16, 16)] = (
                        r0_v[r, pl.ds(c * 16, 16)]
                        + r1_v[r, pl.ds(c * 16, 16)])
                return 0

            lax.fori_loop(0, CCH, body, 0)
            pltpu.sync_copy(r0_v, out_hbm.at[pl.ds(b0, CCH)])

    return k(y, pos0, pos1)


def kernel(hidden_states, w_gate, w1, w3, w2):
    orig_shape = hidden_states.shape
    x = hidden_states.reshape(T, D)
    dest, wp, be_raw = _router(x, w_gate)
    xw = lax.bitcast_convert_type(
        x.astype(jnp.bfloat16).reshape(T, DW, 2), jnp.int32)
    xs_w, wsort = _sc_dispatch(xw, dest.reshape(TK), wp.reshape(TK))
    x_sorted = lax.bitcast_convert_type(
        xs_w, jnp.bfloat16).reshape(NPAD, D)
    h = _ffn1(be_raw[:, 0], x_sorted, w1, w3)
    w_bcast = jnp.broadcast_to(wsort[:, None], (NPAD, 128))
    y = _ffn2(be_raw[:, 0], h, w2, w_bcast)
    final = _sc_combine(y, dest[:, 0], dest[:, 1])
    return final.reshape(orig_shape)


# speedup vs baseline: 1.3226x; 1.0038x over previous
import functools

import jax
import jax.numpy as jnp
from jax import lax
from jax.experimental import pallas as pl
from jax.experimental.pallas import tpu as pltpu
from jax.experimental.pallas import tpu_sc as plsc

E = 8
TOP_K = 2
D = 1024
FF = 3584
T = 2048
TK = T * TOP_K
B = 512
NB = (TK + E * B) // B
NPAD = NB * B
FFB = 1792
NFF = FF // FFB

NW = 32
NC = 2
DW = D // 2


def _router_body(x_ref, wg_ref, dest_ref, wp_ref, be_ref):
    x = x_ref[...]
    wg = wg_ref[...]
    logits = lax.dot_general(x, wg, (((1,), (1,)), ((), ())),
                             preferred_element_type=jnp.float32)
    ii = lax.broadcasted_iota(jnp.int32, (T, E), 1)
    m1 = jnp.max(logits, axis=1, keepdims=True)
    i1 = jnp.min(jnp.where(logits == m1, ii, E), axis=1, keepdims=True)
    masked = jnp.where(ii == i1, -jnp.inf, logits)
    m2 = jnp.max(masked, axis=1, keepdims=True)
    i2 = jnp.min(jnp.where(masked == m2, ii, E), axis=1, keepdims=True)
    wa = 1.0 / (1.0 + jnp.exp(m2 - m1))

    oh0 = (i1 == ii).astype(jnp.float32)
    oh1 = (i2 == ii).astype(jnp.float32)
    rowsum = oh0 + oh1
    TC_ = 512
    rr = lax.broadcasted_iota(jnp.int32, (TC_, TC_), 0)
    cc = lax.broadcasted_iota(jnp.int32, (TC_, TC_), 1)
    lstrict = (cc < rr).astype(jnp.float32)
    carry = jnp.zeros((1, E), jnp.float32)
    parts = []
    for ci in range(T // TC_):
        blk = rowsum[TC_ * ci:TC_ * (ci + 1), :]
        parts.append(carry + lax.dot_general(
            lstrict, blk, (((1,), (0,)), ((), ())),
            preferred_element_type=jnp.float32))
        carry = carry + jnp.sum(blk, axis=0, keepdims=True)
    a_excl = jnp.concatenate(parts, axis=0)
    counts = carry
    padded = jnp.bitwise_and(counts.astype(jnp.int32) + (B - 1),
                             -B).astype(jnp.float32)
    e1 = lax.broadcasted_iota(jnp.int32, (E, E), 0)
    e2 = lax.broadcasted_iota(jnp.int32, (E, E), 1)
    u8 = (e1 <= e2).astype(jnp.float32)
    gend = lax.dot_general(padded, u8, (((1,), (0,)), ((), ())),
                           preferred_element_type=jnp.float32)
    gstart = gend - padded
    base0 = jnp.sum(oh0 * gstart, axis=1, keepdims=True)
    base1 = jnp.sum(oh1 * gstart, axis=1, keepdims=True)
    rank0 = jnp.sum(oh0 * a_excl, axis=1, keepdims=True)
    rank1 = jnp.sum(oh1 * a_excl, axis=1, keepdims=True)
    dest0 = (base0 + rank0).astype(jnp.int32)
    dest1 = (base1 + rank1).astype(jnp.int32)
    dest_ref[...] = jnp.concatenate([dest0, dest1], axis=1)
    wp_ref[...] = jnp.concatenate([wa, 1.0 - wa], axis=1)
    bidx = lax.broadcasted_iota(jnp.int32, (NB, E), 0).astype(jnp.float32)
    ends_b = gend * (1.0 / B)
    be = jnp.sum((bidx >= ends_b).astype(jnp.int32), axis=1, keepdims=True)
    be_ref[...] = be


def _router(x, w_gate):
    return pl.pallas_call(
        _router_body,
        out_shape=(
            jax.ShapeDtypeStruct((T, 2), jnp.int32),
            jax.ShapeDtypeStruct((T, 2), jnp.float32),
            jax.ShapeDtypeStruct((NB, 1), jnp.int32),
        ),
    )(x, w_gate)


PPW = TK // NW


def _sc_dispatch(xw, dest, wf):
    mesh = plsc.VectorSubcoreMesh(core_axis_name="c", subcore_axis_name="s")

    @functools.partial(
        pl.kernel,
        mesh=mesh,
        out_type=(
            jax.ShapeDtypeStruct((NPAD, DW), jnp.int32),
            jax.ShapeDtypeStruct((NPAD,), jnp.float32),
        ),
        scratch_types=[
            pltpu.VMEM((PPW,), jnp.int32),
            pltpu.VMEM((PPW,), jnp.int32),
            pltpu.VMEM((PPW,), jnp.float32),
            pltpu.VMEM((PPW, DW), jnp.int32),
            pltpu.SemaphoreType.DMA,
            pltpu.SemaphoreType.DMA,
            pltpu.SemaphoreType.DMA,
        ],
    )
    def k(xw_hbm, dest_hbm, wf_hbm, xs_hbm, ws_hbm,
          tok_v, dst_v, wf_v, rows_v, s0, s1, s2):
        wid = lax.axis_index("s") * NC + lax.axis_index("c")
        p0 = wid * PPW
        for j in range(PPW // 16):
            v = lax.iota(jnp.int32, 16) + (16 * j)
            tok_v[pl.ds(16 * j, 16)] = lax.shift_right_logical(p0 + v, 1)
        pltpu.sync_copy(dest_hbm.at[pl.ds(p0, PPW)], dst_v)
        pltpu.sync_copy(wf_hbm.at[pl.ds(p0, PPW)], wf_v)
        g = pltpu.async_copy(xw_hbm.at[tok_v], rows_v, s0)
        g.wait()
        st = pltpu.async_copy(rows_v, xs_hbm.at[dst_v], s1)
        sw = pltpu.async_copy(wf_v, ws_hbm.at[dst_v], s2)
        st.wait()
        sw.wait()

    return k(xw, dest, wf)


def _ffn1_body(be_ref, x_ref, w1_ref, w3_ref, h_ref, w1s, w3s):
    b = pl.program_id(1)

    @pl.when(be_ref[b] < E)
    def _():
        changed = jnp.logical_or(
            b == 0, be_ref[b] != be_ref[jnp.maximum(b - 1, 0)])

        @pl.when(changed)
        def _():
            w1s[...] = w1_ref[0].astype(jnp.bfloat16)
            w3s[...] = w3_ref[0].astype(jnp.bfloat16)

        xb = x_ref[...]
        g = lax.dot_general(xb, w1s[...], (((1,), (1,)), ((), ())),
                            preferred_element_type=jnp.float32)
        u = lax.dot_general(xb, w3s[...], (((1,), (1,)), ((), ())),
                            preferred_element_type=jnp.float32)
        sg = 1.0 / (1.0 + jnp.exp(-g))
        h_ref[...] = (g * sg * u).astype(jnp.bfloat16)


def _ffn1(be_raw, x_sorted, w1, w3):
    grid_spec = pltpu.PrefetchScalarGridSpec(
        num_scalar_prefetch=1,
        grid=(NFF, NB),
        in_specs=[
            pl.BlockSpec((B, D), lambda f, b, be: (b, 0)),
            pl.BlockSpec((1, FFB, D),
                         lambda f, b, be: (jnp.minimum(be[b], E - 1), f, 0)),
            pl.BlockSpec((1, FFB, D),
                         lambda f, b, be: (jnp.minimum(be[b], E - 1), f, 0)),
        ],
        out_specs=pl.BlockSpec((B, FFB), lambda f, b, be: (b, f)),
        scratch_shapes=[
            pltpu.VMEM((FFB, D), jnp.bfloat16),
            pltpu.VMEM((FFB, D), jnp.bfloat16),
        ],
    )
    return pl.pallas_call(
        _ffn1_body,
        grid_spec=grid_spec,
        out_shape=jax.ShapeDtypeStruct((NPAD, FF), jnp.bfloat16),
        compiler_params=pltpu.CompilerParams(
            dimension_semantics=("arbitrary", "arbitrary")),
    )(be_raw, x_sorted, w1, w3)


def _ffn2_body(be_ref, h_ref, w2_ref, ws_ref, y_ref, w2s):
    b = pl.program_id(0)

    @pl.when(be_ref[b] < E)
    def _():
        changed = jnp.logical_or(
            b == 0, be_ref[b] != be_ref[jnp.maximum(b - 1, 0)])

        @pl.when(changed)
        def _():
            w2s[...] = w2_ref[0].astype(jnp.bfloat16)

        out = lax.dot_general(h_ref[...], w2s[...],
                              (((1,), (1,)), ((), ())),
                              preferred_element_type=jnp.float32)
        y_ref[...] = out * ws_ref[:, 0:1]


def _ffn2(be_raw, h, w2, w_bcast):
    grid_spec = pltpu.PrefetchScalarGridSpec(
        num_scalar_prefetch=1,
        grid=(NB,),
        in_specs=[
            pl.BlockSpec((B, FF), lambda b, be: (b, 0)),
            pl.BlockSpec((1, D, FF),
                         lambda b, be: (jnp.minimum(be[b], E - 1), 0, 0)),
            pl.BlockSpec((B, 128), lambda b, be: (b, 0)),
        ],
        out_specs=pl.BlockSpec((B, D), lambda b, be: (b, 0)),
        scratch_shapes=[
            pltpu.VMEM((D, FF), jnp.bfloat16),
        ],
    )
    return pl.pallas_call(
        _ffn2_body,
        grid_spec=grid_spec,
        out_shape=jax.ShapeDtypeStruct((NPAD, D), jnp.float32),
        compiler_params=pltpu.CompilerParams(
            dimension_semantics=("arbitrary",)),
    )(be_raw, h, w2, w_bcast)


CCH = 32


def _sc_combine(y, pos0, pos1):
    mesh = plsc.VectorSubcoreMesh(core_axis_name="c", subcore_axis_name="s")
    tok_per_w = T // NW

    @functools.partial(
        pl.kernel,
        mesh=mesh,
        out_type=jax.ShapeDtypeStruct((T, D), jnp.float32),
        scratch_types=[
            pltpu.VMEM((CCH,), jnp.int32),
            pltpu.VMEM((CCH,), jnp.int32),
            pltpu.VMEM((CCH, D), jnp.float32),
            pltpu.VMEM((CCH, D), jnp.float32),
            pltpu.SemaphoreType.DMA,
            pltpu.SemaphoreType.DMA,
        ],
    )
    def k(y_hbm, p0_hbm, p1_hbm, out_hbm, i0_v, i1_v, r0_v, r1_v, s0, s1):
        wid = lax.axis_index("s") * NC + lax.axis_index("c")
        base = wid * tok_per_w
        for t in range(tok_per_w // CCH):
            b0 = base + t * CCH
            pltpu.sync_copy(p0_hbm.at[pl.ds(b0, CCH)], i0_v)
            pltpu.sync_copy(p1_hbm.at[pl.ds(b0, CCH)], i1_v)
            cp0 = pltpu.async_copy(y_hbm.at[i0_v], r0_v, s0)
            cp1 = pltpu.async_copy(y_hbm.at[i1_v], r1_v, s1)
            cp0.wait()
            cp1.wait()

            def body(r, _):
                for c in range(D // 16):
                    r0_v[r, pl.ds(c * 16, 16)] = (
                        r0_v[r, pl.ds(c * 16, 16)]
                        + r1_v[r, pl.ds(c * 16, 16)])
                return 0

            lax.fori_loop(0, CCH, body, 0)
            pltpu.sync_copy(r0_v, out_hbm.at[pl.ds(b0, CCH)])

    return k(y, pos0, pos1)


def kernel(hidden_states, w_gate, w1, w3, w2):
    orig_shape = hidden_states.shape
    x = hidden_states.reshape(T, D)
    dest, wp, be_raw = _router(x, w_gate)
    xw = lax.bitcast_convert_type(
        x.astype(jnp.bfloat16).reshape(T, DW, 2), jnp.int32)
    xs_w, wsort = _sc_dispatch(xw, dest.reshape(TK), wp.reshape(TK))
    x_sorted = lax.bitcast_convert_type(
        xs_w, jnp.bfloat16).reshape(NPAD, D)
    h = _ffn1(be_raw[:, 0], x_sorted, w1, w3)
    w_bcast = jnp.broadcast_to(wsort[:, None], (NPAD, 128))
    y = _ffn2(be_raw[:, 0], h, w2, w_bcast)
    final = _sc_combine(y, dest[:, 0], dest[:, 1])
    return final.reshape(orig_shape)
